# Initial kernel scaffold; baseline (speedup 1.0000x reference)
import jax
import jax.numpy as jnp
from jax.experimental import pallas as pl


def kernel(x1, x2, edge_index, Wd1, bd1, Wg1, bg1, Wd2, bd2, Wg2, bg2):
    raise NotImplementedError("write your pallas kernel here")


# speedup vs baseline: 36.9996x; 36.9996x over previous
import functools

import jax
import jax.numpy as jnp
from jax import lax
from jax.experimental import pallas as pl
from jax.experimental.pallas import tpu as pltpu
from jax.experimental.pallas import tpu_sc as plsc

HIDX = 25453
HPAD = 25600
N2 = 2 * HPAD
E = 1628992
CH = 1024
NTILES = 32
NCH = 50
E_PAD = NTILES * NCH * CH
RPS = N2 // 16
PSHIFT = HPAD - HIDX


def _sc_mesh():
    return plsc.VectorSubcoreMesh(core_axis_name="c", subcore_axis_name="s")


def _k0_body(ei_ref, row2_ref, col2_ref, deg_ref,
             deg_spmem, rawr, rawc, idxr, idxc, ones_v, zb):
    c = lax.axis_index("c")
    s = lax.axis_index("s")
    wid = c * 16 + s

    def zf(i, carry):
        zb[pl.ds(i * 16, 16)] = jnp.zeros((16,), jnp.float32)
        return carry
    lax.fori_loop(0, RPS // 16, zf, 0)
    for l in range(8):
        ones_v[pl.ds(l * 16, 16)] = jnp.ones((16,), jnp.float32)
    pltpu.sync_copy(zb, deg_spmem.at[pl.ds(s * RPS, RPS)])
    plsc.subcore_barrier()

    ebase = wid * (NCH * CH)
    rbase = wid * (NCH * 8)

    def chunk(k, carry):
        base = ebase + k * CH
        pltpu.sync_copy(ei_ref.at[0, pl.ds(base, CH)], rawr)
        pltpu.sync_copy(ei_ref.at[1, pl.ds(base, CH)], rawc)
        for j in range(8):
            for l in range(8):
                off = j * 128 + l * 16
                r = rawr[pl.ds(off, 16)]
                cc = rawc[pl.ds(off, 16)]
                ge = base + off + lax.iota(jnp.int32, 16)
                emask = ge < E
                trash = HIDX + (ge & 63)
                r2 = jnp.where(r >= HIDX, r + PSHIFT, r)
                c2 = jnp.where(cc >= HIDX, cc + PSHIFT, cc)
                idxr[j, pl.ds(l * 16, 16)] = jnp.where(emask, r2, trash)
                idxc[j, pl.ds(l * 16, 16)] = jnp.where(emask, c2, trash)
        for j in range(8):
            pltpu.sync_copy(ones_v, deg_spmem.at[idxr.at[j]], add=True)
        pltpu.sync_copy(idxr, row2_ref.at[pl.ds(rbase + k * 8, 8), :])
        pltpu.sync_copy(idxc, col2_ref.at[pl.ds(rbase + k * 8, 8), :])
        return carry
    lax.fori_loop(0, NCH, chunk, 0)
    plsc.subcore_barrier()
    pltpu.sync_copy(deg_spmem.at[pl.ds(s * RPS, RPS)],
                    deg_ref.at[c, pl.ds(s * RPS, RPS)])


def _k0(ei_pad):
    k = pl.kernel(
        _k0_body,
        out_type=(
            jax.ShapeDtypeStruct((E_PAD // 128, 128), jnp.int32),
            jax.ShapeDtypeStruct((E_PAD // 128, 128), jnp.int32),
            jax.ShapeDtypeStruct((2, N2), jnp.float32),
        ),
        mesh=_sc_mesh(),
        scratch_types=[
            pltpu.VMEM_SHARED((N2,), jnp.float32),
            pltpu.VMEM((CH,), jnp.int32),
            pltpu.VMEM((CH,), jnp.int32),
            pltpu.VMEM((8, 128), jnp.int32),
            pltpu.VMEM((8, 128), jnp.int32),
            pltpu.VMEM((128,), jnp.float32),
            pltpu.VMEM((RPS,), jnp.float32),
        ],
    )
    return k(ei_pad)


def _edge_body(width, scaled_ref, row2_ref, col2_ref, out_ref,
               tmp_spmem, idxr, idxc, msg):
    c = lax.axis_index("c")
    s = lax.axis_index("s")
    wid = c * 16 + s
    pltpu.sync_copy(scaled_ref.at[pl.ds(s * RPS, RPS)],
                    tmp_spmem.at[pl.ds(s * RPS, RPS)])
    plsc.subcore_barrier()

    rbase = wid * (NCH * 8)

    def chunk(k, carry):
        rb = rbase + k * 8
        pltpu.sync_copy(row2_ref.at[pl.ds(rb, 8), :], idxr)
        pltpu.sync_copy(col2_ref.at[pl.ds(rb, 8), :], idxc)
        for j in range(8):
            pltpu.sync_copy(scaled_ref.at[idxr.at[j]], msg.at[j & 1])
            pltpu.sync_copy(msg.at[j & 1], tmp_spmem.at[idxc.at[j]], add=True)
        return carry
    lax.fori_loop(0, NCH, chunk, 0)
    plsc.subcore_barrier()
    pltpu.sync_copy(tmp_spmem.at[pl.ds(s * RPS, RPS)],
                    out_ref.at[c, pl.ds(s * RPS, RPS)])


def _edge_pass(scaled, row2, col2, width):
    k = pl.kernel(
        functools.partial(_edge_body, width),
        out_type=jax.ShapeDtypeStruct((2, N2, width), jnp.float32),
        mesh=_sc_mesh(),
        compiler_params=pltpu.CompilerParams(use_tc_tiling_on_sc=False),
        scratch_types=[
            pltpu.VMEM_SHARED((N2, width), jnp.float32),
            pltpu.VMEM((8, 128), jnp.int32),
            pltpu.VMEM((8, 128), jnp.int32),
            pltpu.VMEM((2, 128, width), jnp.float32),
        ],
    )
    return k(scaled, row2, col2)


def _dense1_body(x_ref, w_ref, b_ref, deg_ref, out_ref, dinv_ref):
    i = pl.program_id(0)
    acc = jnp.dot(x_ref[...], w_ref[...],
                  preferred_element_type=jnp.float32) + b_ref[...][None, :]
    d = deg_ref[...]
    deg = 1.0 + d[:, 0:1] + d[:, 1:2]
    dinv = lax.rsqrt(deg)
    rows = i * 256 + lax.broadcasted_iota(jnp.int32, (256, 1), 0)
    valid = rows < HIDX
    out_ref[...] = jnp.where(valid, dinv * acc, 0.0)
    dinv_ref[...] = jnp.where(valid, dinv, 1.0)


def _dense1(x, w, b, degT_part):
    return pl.pallas_call(
        _dense1_body,
        grid=(HPAD // 256,),
        in_specs=[
            pl.BlockSpec((256, 200), lambda i: (i, 0)),
            pl.BlockSpec((200, 32), lambda i: (0, 0)),
            pl.BlockSpec((32,), lambda i: (0,)),
            pl.BlockSpec((256, 2), lambda i: (i, 0)),
        ],
        out_specs=[
            pl.BlockSpec((256, 32), lambda i: (i, 0)),
            pl.BlockSpec((256, 1), lambda i: (i, 0)),
        ],
        out_shape=[
            jax.ShapeDtypeStruct((HPAD, 32), jnp.float32),
            jax.ShapeDtypeStruct((HPAD, 1), jnp.float32),
        ],
    )(x, w, b, degT_part)


def _dense2_body(t_ref, s_ref, dinv_ref, w_ref, b_ref, out_ref):
    i = pl.program_id(0)
    t = t_ref[0] + t_ref[1] - s_ref[...]
    a = dinv_ref[...] * t
    a = jnp.where(a >= 0, a, 0.01 * a)
    b = jnp.where(i < 100, b_ref[0:1, :], b_ref[1:2, :])
    c2 = jnp.dot(a, w_ref[0],
                 preferred_element_type=jnp.float32) + b
    sc2 = dinv_ref[...] * c2
    rows = i * 256 + lax.broadcasted_iota(jnp.int32, (256, 1), 0)
    valid = (rows < HIDX) | ((rows >= HPAD) & (rows < HPAD + HIDX))
    out_ref[...] = jnp.where(valid, sc2, 0.0)


def _dense2(tmp1, scaled1, dinv, w2s, b2s):
    return pl.pallas_call(
        _dense2_body,
        grid=(N2 // 256,),
        in_specs=[
            pl.BlockSpec((2, 256, 32), lambda i: (0, i, 0)),
            pl.BlockSpec((256, 32), lambda i: (i, 0)),
            pl.BlockSpec((256, 1), lambda i: (i, 0)),
            pl.BlockSpec((1, 32, 16), lambda i: (i // 100, 0, 0)),
            pl.BlockSpec((2, 16), lambda i: (0, 0)),
        ],
        out_specs=pl.BlockSpec((256, 16), lambda i: (i, 0)),
        out_shape=jax.ShapeDtypeStruct((N2, 16), jnp.float32),
    )(tmp1, scaled1, dinv, w2s, b2s)


def _softmax_body(t_ref, s_ref, dinv_ref, out_ref):
    t = t_ref[0] + t_ref[1] - s_ref[...]
    a = dinv_ref[...] * t
    a = jnp.where(a >= 0, a, 0.01 * a)
    lane = lax.broadcasted_iota(jnp.int32, (256, 16), 1)
    valid = lane < 7
    m = jnp.max(jnp.where(valid, a, -jnp.inf), axis=1, keepdims=True)
    e = jnp.where(valid, jnp.exp(a - m), 0.0)
    out_ref[...] = e / jnp.sum(e, axis=1, keepdims=True)


def _softmax(tmp2, scaled2, dinv):
    return pl.pallas_call(
        _softmax_body,
        grid=(HPAD // 256,),
        in_specs=[
            pl.BlockSpec((2, 256, 16), lambda i: (0, i, 0)),
            pl.BlockSpec((256, 16), lambda i: (i, 0)),
            pl.BlockSpec((256, 1), lambda i: (i, 0)),
        ],
        out_specs=pl.BlockSpec((256, 16), lambda i: (i, 0)),
        out_shape=jax.ShapeDtypeStruct((HPAD, 16), jnp.float32),
    )(tmp2, scaled2, dinv)


def kernel(x1, x2, edge_index, Wd1, bd1, Wg1, bg1, Wd2, bd2, Wg2, bg2):
    ei = edge_index.astype(jnp.int32)
    ei_pad = jnp.pad(ei, ((0, 0), (0, E_PAD - E)))

    row2, col2, deg_parts = _k0(ei_pad)
    degT = deg_parts.T

    wd1 = jnp.pad(Wd1, ((0, 0), (0, 2)))
    wg1 = jnp.pad(Wg1, ((0, 0), (0, 2)))
    b1d = jnp.pad(bd1, (0, 2))
    b1g = jnp.pad(bg1, (0, 2))
    s1a, dinva = _dense1(x1, wd1, b1d, degT[:HPAD])
    s1b, dinvb = _dense1(x2, wg1, b1g, degT[HPAD:])
    scaled1 = jnp.concatenate([s1a, s1b], axis=0)
    dinv = jnp.concatenate([dinva, dinvb], axis=0)

    tmp1 = _edge_pass(scaled1, row2, col2, 32)

    w2s = jnp.stack([jnp.pad(Wd2, ((0, 2), (0, 9))),
                     jnp.pad(Wg2, ((0, 2), (0, 9)))])
    b2s = jnp.stack([jnp.pad(bd2, (0, 9)), jnp.pad(bg2, (0, 9))])
    scaled2 = _dense2(tmp1, scaled1, dinv, w2s, b2s)

    tmp2 = _edge_pass(scaled2, row2, col2, 16)

    out = _softmax(tmp2, scaled2, dinv)
    return out[:HIDX, :7]


# speedup vs baseline: 54.6710x; 1.4776x over previous
import functools

import jax
import jax.numpy as jnp
from jax import lax
from jax.experimental import pallas as pl
from jax.experimental.pallas import tpu as pltpu
from jax.experimental.pallas import tpu_sc as plsc

HIDX = 25453
HPAD = 25600
N2 = 2 * HPAD
E = 1628992
CH = 1024
NTILES = 32
NCH = 50
E_PAD = NTILES * NCH * CH
RPS = N2 // 16
PSHIFT = HPAD - HIDX


def _sc_mesh():
    return plsc.VectorSubcoreMesh(core_axis_name="c", subcore_axis_name="s")


def _k0_body(ei_ref, row2_ref, col2_ref, deg_ref,
             deg_spmem, rawr, rawc, idxr, idxc, ones_v, zb):
    c = lax.axis_index("c")
    s = lax.axis_index("s")
    wid = c * 16 + s

    def zf(i, carry):
        zb[pl.ds(i * 16, 16)] = jnp.zeros((16,), jnp.float32)
        return carry
    lax.fori_loop(0, RPS // 16, zf, 0)
    for l in range(8):
        ones_v[pl.ds(l * 16, 16)] = jnp.ones((16,), jnp.float32)
    pltpu.sync_copy(zb, deg_spmem.at[pl.ds(s * RPS, RPS)])
    plsc.subcore_barrier()

    ebase = wid * (NCH * CH)
    rbase = wid * (NCH * 8)

    def chunk(k, carry):
        base = ebase + k * CH
        pltpu.sync_copy(ei_ref.at[0, pl.ds(base, CH)], rawr)
        pltpu.sync_copy(ei_ref.at[1, pl.ds(base, CH)], rawc)
        for j in range(8):
            for l in range(8):
                off = j * 128 + l * 16
                r = rawr[pl.ds(off, 16)]
                cc = rawc[pl.ds(off, 16)]
                ge = base + off + lax.iota(jnp.int32, 16)
                emask = ge < E
                trash = HIDX + (ge & 63)
                r2 = jnp.where(r >= HIDX, r + PSHIFT, r)
                c2 = jnp.where(cc >= HIDX, cc + PSHIFT, cc)
                idxr[j, pl.ds(l * 16, 16)] = jnp.where(emask, r2, trash)
                idxc[j, pl.ds(l * 16, 16)] = jnp.where(emask, c2, trash)
        for j in range(8):
            pltpu.sync_copy(ones_v, deg_spmem.at[idxr.at[j]], add=True)
        pltpu.sync_copy(idxr, row2_ref.at[pl.ds(rbase + k * 8, 8), :])
        pltpu.sync_copy(idxc, col2_ref.at[pl.ds(rbase + k * 8, 8), :])
        return carry
    lax.fori_loop(0, NCH, chunk, 0)
    plsc.subcore_barrier()
    pltpu.sync_copy(deg_spmem.at[pl.ds(s * RPS, RPS)],
                    deg_ref.at[c, pl.ds(s * RPS, RPS)])


def _k0(ei_pad):
    k = pl.kernel(
        _k0_body,
        out_type=(
            jax.ShapeDtypeStruct((E_PAD // 128, 128), jnp.int32),
            jax.ShapeDtypeStruct((E_PAD // 128, 128), jnp.int32),
            jax.ShapeDtypeStruct((2, N2), jnp.float32),
        ),
        mesh=_sc_mesh(),
        scratch_types=[
            pltpu.VMEM_SHARED((N2,), jnp.float32),
            pltpu.VMEM((CH,), jnp.int32),
            pltpu.VMEM((CH,), jnp.int32),
            pltpu.VMEM((8, 128), jnp.int32),
            pltpu.VMEM((8, 128), jnp.int32),
            pltpu.VMEM((128,), jnp.float32),
            pltpu.VMEM((RPS,), jnp.float32),
        ],
    )
    return k(ei_pad)


def _edge_body(width, nslots, scaled_ref, row2_ref, col2_ref, out_ref,
               tmp_spmem, idxr, idxc, msg, gsem, ssem, isem):
    c = lax.axis_index("c")
    s = lax.axis_index("s")
    wid = c * 16 + s
    pltpu.sync_copy(scaled_ref.at[pl.ds(s * RPS, RPS)],
                    tmp_spmem.at[pl.ds(s * RPS, RPS)])
    plsc.subcore_barrier()

    rbase = wid * (NCH * 8)

    def idx_issue(k, buf):
        rb = rbase + k * 8
        pltpu.async_copy(row2_ref.at[pl.ds(rb, 8), :], idxr.at[buf], isem)
        pltpu.async_copy(col2_ref.at[pl.ds(rb, 8), :], idxc.at[buf], isem)

    def idx_wait(k, buf):
        rb = rbase + k * 8
        pltpu.make_async_copy(row2_ref.at[pl.ds(rb, 8), :], idxr.at[buf],
                              isem).wait()
        pltpu.make_async_copy(col2_ref.at[pl.ds(rb, 8), :], idxc.at[buf],
                              isem).wait()

    def gth(buf, j, sl):
        return scaled_ref.at[idxr.at[buf, j]], msg.at[sl], gsem.at[sl]

    def sct(buf, j, sl):
        return msg.at[sl], tmp_spmem.at[idxc.at[buf, j]], ssem.at[sl]

    idx_issue(0, 0)

    def chunk(k, carry):
        buf = k & 1
        idx_wait(k, buf)
        for j in range(nslots):
            src, dst, sem = sct(1 - buf, j + 8 - nslots, j)

            @pl.when(k > 0)
            def _():
                pltpu.make_async_copy(src, dst, sem).wait()

        @pl.when(k + 1 < NCH)
        def _():
            idx_issue(k + 1, (k + 1) & 1)
        for j in range(nslots):
            src, dst, sem = gth(buf, j, j)
            pltpu.async_copy(src, dst, sem)
        for j in range(8):
            sl = j % nslots
            if j >= nslots:
                src, dst, sem = sct(buf, j - nslots, sl)
                pltpu.make_async_copy(src, dst, sem).wait()
                src, dst, sem = gth(buf, j, sl)
                pltpu.async_copy(src, dst, sem)
            src, dst, sem = gth(buf, j, sl)
            pltpu.make_async_copy(src, dst, sem).wait()
            src, dst, sem = sct(buf, j, sl)
            pltpu.async_copy(src, dst, sem, add=True)
        return carry
    lax.fori_loop(0, NCH, chunk, 0)
    buf = (NCH - 1) & 1
    for j in range(nslots):
        src, dst, sem = sct(buf, j + 8 - nslots, j)
        pltpu.make_async_copy(src, dst, sem).wait()
    plsc.subcore_barrier()
    pltpu.sync_copy(tmp_spmem.at[pl.ds(s * RPS, RPS)],
                    out_ref.at[c, pl.ds(s * RPS, RPS)])


def _edge_pass(scaled, row2, col2, width):
    nslots = 4 if width == 32 else 8
    k = pl.kernel(
        functools.partial(_edge_body, width, nslots),
        out_type=jax.ShapeDtypeStruct((2, N2, width), jnp.float32),
        mesh=_sc_mesh(),
        compiler_params=pltpu.CompilerParams(use_tc_tiling_on_sc=False),
        scratch_types=[
            pltpu.VMEM_SHARED((N2, width), jnp.float32),
            pltpu.VMEM((2, 8, 128), jnp.int32),
            pltpu.VMEM((2, 8, 128), jnp.int32),
            pltpu.VMEM((nslots, 128, width), jnp.float32),
            pltpu.SemaphoreType.DMA((nslots,)),
            pltpu.SemaphoreType.DMA((nslots,)),
            pltpu.SemaphoreType.DMA,
        ],
    )
    return k(scaled, row2, col2)


def _dense1_body(x_ref, w_ref, b_ref, deg_ref, out_ref, dinv_ref):
    i = pl.program_id(0)
    acc = jnp.dot(x_ref[...], w_ref[...],
                  preferred_element_type=jnp.float32) + b_ref[...][None, :]
    d = deg_ref[...]
    deg = 1.0 + d[:, 0:1] + d[:, 1:2]
    dinv = lax.rsqrt(deg)
    rows = i * 256 + lax.broadcasted_iota(jnp.int32, (256, 1), 0)
    valid = rows < HIDX
    out_ref[...] = jnp.where(valid, dinv * acc, 0.0)
    dinv_ref[...] = jnp.where(valid, dinv, 1.0)


def _dense1(x, w, b, degT_part):
    return pl.pallas_call(
        _dense1_body,
        grid=(HPAD // 256,),
        in_specs=[
            pl.BlockSpec((256, 200), lambda i: (i, 0)),
            pl.BlockSpec((200, 32), lambda i: (0, 0)),
            pl.BlockSpec((32,), lambda i: (0,)),
            pl.BlockSpec((256, 2), lambda i: (i, 0)),
        ],
        out_specs=[
            pl.BlockSpec((256, 32), lambda i: (i, 0)),
            pl.BlockSpec((256, 1), lambda i: (i, 0)),
        ],
        out_shape=[
            jax.ShapeDtypeStruct((HPAD, 32), jnp.float32),
            jax.ShapeDtypeStruct((HPAD, 1), jnp.float32),
        ],
    )(x, w, b, degT_part)


def _dense2_body(t_ref, s_ref, dinv_ref, w_ref, b_ref, out_ref):
    i = pl.program_id(0)
    t = t_ref[0] + t_ref[1] - s_ref[...]
    a = dinv_ref[...] * t
    a = jnp.where(a >= 0, a, 0.01 * a)
    b = jnp.where(i < 100, b_ref[0:1, :], b_ref[1:2, :])
    c2 = jnp.dot(a, w_ref[0],
                 preferred_element_type=jnp.float32) + b
    sc2 = dinv_ref[...] * c2
    rows = i * 256 + lax.broadcasted_iota(jnp.int32, (256, 1), 0)
    valid = (rows < HIDX) | ((rows >= HPAD) & (rows < HPAD + HIDX))
    out_ref[...] = jnp.where(valid, sc2, 0.0)


def _dense2(tmp1, scaled1, dinv, w2s, b2s):
    return pl.pallas_call(
        _dense2_body,
        grid=(N2 // 256,),
        in_specs=[
            pl.BlockSpec((2, 256, 32), lambda i: (0, i, 0)),
            pl.BlockSpec((256, 32), lambda i: (i, 0)),
            pl.BlockSpec((256, 1), lambda i: (i, 0)),
            pl.BlockSpec((1, 32, 16), lambda i: (i // 100, 0, 0)),
            pl.BlockSpec((2, 16), lambda i: (0, 0)),
        ],
        out_specs=pl.BlockSpec((256, 16), lambda i: (i, 0)),
        out_shape=jax.ShapeDtypeStruct((N2, 16), jnp.float32),
    )(tmp1, scaled1, dinv, w2s, b2s)


def _softmax_body(t_ref, s_ref, dinv_ref, out_ref):
    t = t_ref[0] + t_ref[1] - s_ref[...]
    a = dinv_ref[...] * t
    a = jnp.where(a >= 0, a, 0.01 * a)
    lane = lax.broadcasted_iota(jnp.int32, (256, 16), 1)
    valid = lane < 7
    m = jnp.max(jnp.where(valid, a, -jnp.inf), axis=1, keepdims=True)
    e = jnp.where(valid, jnp.exp(a - m), 0.0)
    out_ref[...] = e / jnp.sum(e, axis=1, keepdims=True)


def _softmax(tmp2, scaled2, dinv):
    return pl.pallas_call(
        _softmax_body,
        grid=(HPAD // 256,),
        in_specs=[
            pl.BlockSpec((2, 256, 16), lambda i: (0, i, 0)),
            pl.BlockSpec((256, 16), lambda i: (i, 0)),
            pl.BlockSpec((256, 1), lambda i: (i, 0)),
        ],
        out_specs=pl.BlockSpec((256, 16), lambda i: (i, 0)),
        out_shape=jax.ShapeDtypeStruct((HPAD, 16), jnp.float32),
    )(tmp2, scaled2, dinv)


def kernel(x1, x2, edge_index, Wd1, bd1, Wg1, bg1, Wd2, bd2, Wg2, bg2):
    ei = edge_index.astype(jnp.int32)
    ei_pad = jnp.pad(ei, ((0, 0), (0, E_PAD - E)))

    row2, col2, deg_parts = _k0(ei_pad)
    degT = deg_parts.T

    wd1 = jnp.pad(Wd1, ((0, 0), (0, 2)))
    wg1 = jnp.pad(Wg1, ((0, 0), (0, 2)))
    b1d = jnp.pad(bd1, (0, 2))
    b1g = jnp.pad(bg1, (0, 2))
    s1a, dinva = _dense1(x1, wd1, b1d, degT[:HPAD])
    s1b, dinvb = _dense1(x2, wg1, b1g, degT[HPAD:])
    scaled1 = jnp.concatenate([s1a, s1b], axis=0)
    dinv = jnp.concatenate([dinva, dinvb], axis=0)

    tmp1 = _edge_pass(scaled1, row2, col2, 32)

    w2s = jnp.stack([jnp.pad(Wd2, ((0, 2), (0, 9))),
                     jnp.pad(Wg2, ((0, 2), (0, 9)))])
    b2s = jnp.stack([jnp.pad(bd2, (0, 9)), jnp.pad(bg2, (0, 9))])
    scaled2 = _dense2(tmp1, scaled1, dinv, w2s, b2s)

    tmp2 = _edge_pass(scaled2, row2, col2, 16)

    out = _softmax(tmp2, scaled2, dinv)
    return out[:HIDX, :7]


# speedup vs baseline: 57.4136x; 1.0502x over previous
import functools

import jax
import jax.numpy as jnp
from jax import lax
from jax.experimental import pallas as pl
from jax.experimental.pallas import tpu as pltpu
from jax.experimental.pallas import tpu_sc as plsc

HIDX = 25453
HPAD = 25600
N2 = 2 * HPAD
E = 1628992
KCH = 1024
NTILES = 32
KNCH = 50
E_PAD = NTILES * KNCH * KCH
EP128 = E_PAD // 128
RPS = N2 // 16
PSHIFT = HPAD - HIDX

ECH = 512
DEPTH = 6
IDEPTH = 2 * DEPTH
LAG = 2


def _sc_mesh():
    return plsc.VectorSubcoreMesh(core_axis_name="c", subcore_axis_name="s")


def _k0_body(ei_ref, row2_ref, col2_ref, deg_ref,
             deg_spmem, rawr, rawc, idxlo, idxhi, idxc, ones_v, zb,
             lsem, osem, dsem):
    c = lax.axis_index("c")
    s = lax.axis_index("s")
    wid = c * 16 + s

    def zf(i, carry):
        zb[pl.ds(i * 16, 16)] = jnp.zeros((16,), jnp.float32)
        return carry
    lax.fori_loop(0, RPS // 16, zf, 0)
    for l in range(8):
        for q in range(8):
            ones_v[l, pl.ds(q * 16, 16)] = jnp.ones((16,), jnp.float32)
    pltpu.sync_copy(zb, deg_spmem.at[pl.ds(s * RPS, RPS)])
    plsc.subcore_barrier()

    ebase = wid * (KNCH * KCH)
    rbase = wid * (KNCH * 8)

    def loads_issue(k, d):
        base = ebase + k * KCH
        pltpu.async_copy(ei_ref.at[0, pl.ds(base, KCH)], rawr.at[d], lsem)
        pltpu.async_copy(ei_ref.at[1, pl.ds(base, KCH)], rawc.at[d], lsem)

    def loads_wait(k, d):
        base = ebase + k * KCH
        pltpu.make_async_copy(ei_ref.at[0, pl.ds(base, KCH)], rawr.at[d],
                              lsem).wait()
        pltpu.make_async_copy(ei_ref.at[1, pl.ds(base, KCH)], rawc.at[d],
                              lsem).wait()

    def outs(k, d):
        rb = rbase + k * 8
        yield idxlo.at[d], row2_ref.at[pl.ds(rb, 8), :]
        yield idxhi.at[d], row2_ref.at[pl.ds(EP128 + rb, 8), :]
        yield idxc.at[d], col2_ref.at[pl.ds(rb, 8), :]

    loads_issue(0, 0)

    def chunk(k, carry):
        d = k & 1

        @pl.when(k + 1 < KNCH)
        def _():
            loads_issue(k + 1, 1 - d)
        loads_wait(k, d)

        @pl.when(k >= 2)
        def _():
            for src, dst in outs(k - 2, d):
                pltpu.make_async_copy(src, dst, osem).wait()
            for j in range(8):
                pltpu.make_async_copy(ones_v.at[j],
                                      deg_spmem.at[idxlo.at[d, j]],
                                      dsem).wait()
        base = ebase + k * KCH
        for j in range(8):
            for l in range(8):
                off = j * 128 + l * 16
                r = rawr[d, pl.ds(off, 16)]
                cc = rawc[d, pl.ds(off, 16)]
                ge = base + off + lax.iota(jnp.int32, 16)
                emask = ge < E
                trash = HIDX + (ge & 63)
                r2 = jnp.where(r >= HIDX, r + PSHIFT, r)
                c2 = jnp.where(cc >= HIDX, cc + PSHIFT, cc)
                r2 = jnp.where(emask, r2, trash)
                idxlo[d, j, pl.ds(l * 16, 16)] = r2
                idxhi[d, j, pl.ds(l * 16, 16)] = r2 + N2
                idxc[d, j, pl.ds(l * 16, 16)] = jnp.where(emask, c2, trash)
        for src, dst in outs(k, d):
            pltpu.async_copy(src, dst, osem)
        for j in range(8):
            pltpu.async_copy(ones_v.at[j], deg_spmem.at[idxlo.at[d, j]],
                             dsem, add=True)
        return carry
    lax.fori_loop(0, KNCH, chunk, 0)
    for k in (KNCH - 2, KNCH - 1):
        d = k & 1
        for src, dst in outs(k, d):
            pltpu.make_async_copy(src, dst, osem).wait()
        for j in range(8):
            pltpu.make_async_copy(ones_v.at[j], deg_spmem.at[idxlo.at[d, j]],
                                  dsem).wait()
    plsc.subcore_barrier()
    pltpu.sync_copy(deg_spmem.at[pl.ds(s * RPS, RPS)],
                    deg_ref.at[c, pl.ds(s * RPS, RPS)])


def _k0(ei_pad):
    k = pl.kernel(
        _k0_body,
        out_type=(
            jax.ShapeDtypeStruct((2 * EP128, 128), jnp.int32),
            jax.ShapeDtypeStruct((EP128, 128), jnp.int32),
            jax.ShapeDtypeStruct((2, N2), jnp.float32),
        ),
        mesh=_sc_mesh(),
        compiler_params=pltpu.CompilerParams(use_tc_tiling_on_sc=False),
        scratch_types=[
            pltpu.VMEM_SHARED((N2,), jnp.float32),
            pltpu.VMEM((2, KCH), jnp.int32),
            pltpu.VMEM((2, KCH), jnp.int32),
            pltpu.VMEM((2, 8, 128), jnp.int32),
            pltpu.VMEM((2, 8, 128), jnp.int32),
            pltpu.VMEM((2, 8, 128), jnp.int32),
            pltpu.VMEM((8, 128), jnp.float32),
            pltpu.VMEM((RPS,), jnp.float32),
            pltpu.SemaphoreType.DMA,
            pltpu.SemaphoreType.DMA,
            pltpu.SemaphoreType.DMA,
        ],
    )
    return k(ei_pad)


DEPTH = 16
LAG = 4
IDEPTH = 4


def _edge_body(mode, nch, scaled_ref, row2_ref, col2_ref, out_ref,
               tmp_spmem, idxr, idxc, msg, gsem, ssem, isem):
    c = lax.axis_index("c")
    s = lax.axis_index("s")
    if mode == "half":
        rrow = c * EP128 + s * (nch * 8)
        rcol = s * (nch * 8)
        stage = c * N2 + s * RPS
    else:
        wid = c * 16 + s
        rrow = wid * (nch * 8)
        rcol = rrow
        stage = s * RPS
    pltpu.sync_copy(scaled_ref.at[pl.ds(stage, RPS)],
                    tmp_spmem.at[pl.ds(s * RPS, RPS)])
    plsc.subcore_barrier()

    def idx_issue(k):
        di = k % IDEPTH
        pltpu.async_copy(row2_ref.at[pl.ds(rrow + k * 8, 8), :],
                         idxr.at[di], isem)
        pltpu.async_copy(col2_ref.at[pl.ds(rcol + k * 8, 8), :],
                         idxc.at[di], isem)

    def idx_wait(k):
        di = k % IDEPTH
        pltpu.make_async_copy(row2_ref.at[pl.ds(rrow + k * 8, 8), :],
                              idxr.at[di], isem).wait()
        pltpu.make_async_copy(col2_ref.at[pl.ds(rcol + k * 8, 8), :],
                              idxc.at[di], isem).wait()

    def swait():
        pltpu.make_async_copy(msg.at[0], tmp_spmem.at[idxc.at[0, 0]],
                              ssem).wait()

    def gwait():
        pltpu.make_async_copy(scaled_ref.at[idxr.at[0, 0]], msg.at[0],
                              gsem).wait()

    idx_issue(0)
    idx_issue(1)

    def chunk(k, carry):
        idx_wait(k)
        di = k % IDEPTH
        for j in range(8):
            d = j + 8 * (k % 2)

            @pl.when(k >= 2)
            def _():
                swait()
            pltpu.async_copy(scaled_ref.at[idxr.at[di, j]], msg.at[d], gsem)

            def sct_issue():
                g2 = k * 8 + j - LAG
                d2 = g2 % DEPTH
                k2 = g2 // 8
                pltpu.make_async_copy(scaled_ref.at[idxr.at[0, 0]],
                                      msg.at[0], gsem).wait()
                pltpu.async_copy(msg.at[d2],
                                 tmp_spmem.at[idxc.at[k2 % IDEPTH, g2 % 8]],
                                 ssem, add=True)
            if j >= LAG:
                sct_issue()
            else:
                @pl.when(k >= 1)
                def _():
                    sct_issue()
        idx_issue(k + 2)
        return carry
    lax.fori_loop(0, nch - 2, chunk, 0)
    for k in (nch - 2, nch - 1):
        idx_wait(k)
        di = k % IDEPTH
        for j in range(8):
            d = j + 8 * (k % 2)
            swait()
            pltpu.async_copy(scaled_ref.at[idxr.at[di, j]], msg.at[d], gsem)
            g2 = k * 8 + j - LAG
            gwait()
            pltpu.async_copy(msg.at[g2 % DEPTH],
                             tmp_spmem.at[idxc.at[(g2 // 8) % IDEPTH,
                                                  g2 % 8]],
                             ssem, add=True)
    for g in range(nch * 8 - LAG, nch * 8):
        gwait()
        pltpu.async_copy(msg.at[g % DEPTH],
                         tmp_spmem.at[idxc.at[(g // 8) % IDEPTH, g % 8]],
                         ssem, add=True)
    for _ in range(DEPTH):
        swait()
    plsc.subcore_barrier()
    pltpu.sync_copy(tmp_spmem.at[pl.ds(s * RPS, RPS)],
                    out_ref.at[c, pl.ds(s * RPS, RPS)])


def _edge_pass(scaled, row2, col2, mode):
    nch = (E_PAD // 16 // KCH) if mode == "half" else (E_PAD // 32 // KCH)
    k = pl.kernel(
        functools.partial(_edge_body, mode, nch),
        out_type=jax.ShapeDtypeStruct((2, N2, 16), jnp.float32),
        mesh=_sc_mesh(),
        compiler_params=pltpu.CompilerParams(use_tc_tiling_on_sc=False),
        scratch_types=[
            pltpu.VMEM_SHARED((N2, 16), jnp.float32),
            pltpu.VMEM((IDEPTH, 8, 128), jnp.int32),
            pltpu.VMEM((IDEPTH, 8, 128), jnp.int32),
            pltpu.VMEM((DEPTH, 128, 16), jnp.float32),
            pltpu.SemaphoreType.DMA,
            pltpu.SemaphoreType.DMA,
            pltpu.SemaphoreType.DMA,
        ],
    )
    return k(scaled, row2, col2)


def _dense1_body(x_ref, w_ref, b_ref, deg_ref, out0_ref, out1_ref, dinv_ref):
    i = pl.program_id(0)
    acc = jnp.dot(x_ref[...], w_ref[...],
                  preferred_element_type=jnp.float32) + b_ref[...][None, :]
    d = deg_ref[...]
    deg = 1.0 + d[:, 0:1] + d[:, 1:2]
    dinv = lax.rsqrt(deg)
    rows = i * 256 + lax.broadcasted_iota(jnp.int32, (256, 1), 0)
    valid = rows < HIDX
    sc = jnp.where(valid, dinv * acc, 0.0)
    out0_ref[...] = sc[:, :16]
    out1_ref[...] = sc[:, 16:]
    dinv_ref[...] = jnp.where(valid, dinv, 1.0)


def _dense1(x, w, b, degT_part):
    return pl.pallas_call(
        _dense1_body,
        grid=(HPAD // 256,),
        in_specs=[
            pl.BlockSpec((256, 200), lambda i: (i, 0)),
            pl.BlockSpec((200, 32), lambda i: (0, 0)),
            pl.BlockSpec((32,), lambda i: (0,)),
            pl.BlockSpec((256, 2), lambda i: (i, 0)),
        ],
        out_specs=[
            pl.BlockSpec((256, 16), lambda i: (i, 0)),
            pl.BlockSpec((256, 16), lambda i: (i, 0)),
            pl.BlockSpec((256, 1), lambda i: (i, 0)),
        ],
        out_shape=[
            jax.ShapeDtypeStruct((HPAD, 16), jnp.float32),
            jax.ShapeDtypeStruct((HPAD, 16), jnp.float32),
            jax.ShapeDtypeStruct((HPAD, 1), jnp.float32),
        ],
    )(x, w, b, degT_part)


def _dense2_body(t_ref, dinv_ref, w_ref, b_ref, out_ref):
    i = pl.program_id(0)
    t = jnp.concatenate([t_ref[0], t_ref[1]], axis=1)
    a = dinv_ref[...] * t
    a = jnp.where(a >= 0, a, 0.01 * a)
    b = jnp.where(i < 100, b_ref[0:1, :], b_ref[1:2, :])
    c2 = jnp.dot(a, w_ref[0],
                 preferred_element_type=jnp.float32) + b
    sc2 = dinv_ref[...] * c2
    rows = i * 256 + lax.broadcasted_iota(jnp.int32, (256, 1), 0)
    valid = (rows < HIDX) | ((rows >= HPAD) & (rows < HPAD + HIDX))
    out_ref[...] = jnp.where(valid, sc2, 0.0)


def _dense2(tmp1, dinv, w2s, b2s):
    return pl.pallas_call(
        _dense2_body,
        grid=(N2 // 256,),
        in_specs=[
            pl.BlockSpec((2, 256, 16), lambda i: (0, i, 0)),
            pl.BlockSpec((256, 1), lambda i: (i, 0)),
            pl.BlockSpec((1, 32, 16), lambda i: (i // 100, 0, 0)),
            pl.BlockSpec((2, 16), lambda i: (0, 0)),
        ],
        out_specs=pl.BlockSpec((256, 16), lambda i: (i, 0)),
        out_shape=jax.ShapeDtypeStruct((N2, 16), jnp.float32),
    )(tmp1, dinv, w2s, b2s)


def _softmax_body(t_ref, s_ref, dinv_ref, out_ref):
    t = t_ref[0] + t_ref[1] - s_ref[...]
    a = dinv_ref[...] * t
    a = jnp.where(a >= 0, a, 0.01 * a)
    lane = lax.broadcasted_iota(jnp.int32, (256, 16), 1)
    valid = lane < 7
    m = jnp.max(jnp.where(valid, a, -jnp.inf), axis=1, keepdims=True)
    e = jnp.where(valid, jnp.exp(a - m), 0.0)
    out_ref[...] = e / jnp.sum(e, axis=1, keepdims=True)


def _softmax(tmp2, scaled2, dinv):
    return pl.pallas_call(
        _softmax_body,
        grid=(HPAD // 256,),
        in_specs=[
            pl.BlockSpec((2, 256, 16), lambda i: (0, i, 0)),
            pl.BlockSpec((256, 16), lambda i: (i, 0)),
            pl.BlockSpec((256, 1), lambda i: (i, 0)),
        ],
        out_specs=pl.BlockSpec((256, 16), lambda i: (i, 0)),
        out_shape=jax.ShapeDtypeStruct((HPAD, 16), jnp.float32),
    )(tmp2, scaled2, dinv)


def kernel(x1, x2, edge_index, Wd1, bd1, Wg1, bg1, Wd2, bd2, Wg2, bg2):
    ei = edge_index.astype(jnp.int32)
    ei_pad = jnp.pad(ei, ((0, 0), (0, E_PAD - E)))

    row2, col2, deg_parts = _k0(ei_pad)
    degT = deg_parts.T

    wd1 = jnp.pad(Wd1, ((0, 0), (0, 2)))
    wg1 = jnp.pad(Wg1, ((0, 0), (0, 2)))
    b1d = jnp.pad(bd1, (0, 2))
    b1g = jnp.pad(bg1, (0, 2))
    h0a, h1a, dinva = _dense1(x1, wd1, b1d, degT[:HPAD])
    h0b, h1b, dinvb = _dense1(x2, wg1, b1g, degT[HPAD:])
    scaled1 = jnp.concatenate([h0a, h0b, h1a, h1b], axis=0)
    dinv = jnp.concatenate([dinva, dinvb], axis=0)

    tmp1 = _edge_pass(scaled1, row2, col2, "half")

    w2s = jnp.stack([jnp.pad(Wd2, ((0, 2), (0, 9))),
                     jnp.pad(Wg2, ((0, 2), (0, 9)))])
    b2s = jnp.stack([jnp.pad(bd2, (0, 9)), jnp.pad(bg2, (0, 9))])
    scaled2 = _dense2(tmp1, dinv, w2s, b2s)

    tmp2 = _edge_pass(scaled2, row2, col2, "split")

    out = _softmax(tmp2, scaled2, dinv)
    return out[:HIDX, :7]


# speedup vs baseline: 65.6652x; 1.1437x over previous
import functools

import jax
import jax.numpy as jnp
from jax import lax
from jax.experimental import pallas as pl
from jax.experimental.pallas import tpu as pltpu
from jax.experimental.pallas import tpu_sc as plsc

HIDX = 25453
HPAD = 25600
N2 = 2 * HPAD
E = 1628992
KCH = 1024
NTILES = 32
KNCH = 50
E_PAD = NTILES * KNCH * KCH
EP128 = E_PAD // 128
RPS = N2 // 16
PSHIFT = HPAD - HIDX

ECH = 512
DEPTH = 6
IDEPTH = 2 * DEPTH
LAG = 2


def _sc_mesh():
    return plsc.VectorSubcoreMesh(core_axis_name="c", subcore_axis_name="s")


def _k0_body(ei_ref, row2_ref, col2_ref, deg_ref,
             deg_spmem, rawr, rawc, idxlo, idxc, ones_v, zb,
             lsem, osem, dsem):
    c = lax.axis_index("c")
    s = lax.axis_index("s")
    wid = c * 16 + s

    def zf(i, carry):
        zb[pl.ds(i * 16, 16)] = jnp.zeros((16,), jnp.float32)
        return carry
    lax.fori_loop(0, RPS // 16, zf, 0)
    for l in range(8):
        for q in range(8):
            ones_v[l, pl.ds(q * 16, 16)] = jnp.ones((16,), jnp.float32)
    pltpu.sync_copy(zb, deg_spmem.at[pl.ds(s * RPS, RPS)])
    plsc.subcore_barrier()

    ebase = wid * (KNCH * KCH)
    rbase = wid * (KNCH * 8)

    LASTC = E // KCH
    TAIL = E - LASTC * KCH

    def loads_issue(k, d):
        base = ebase + k * KCH
        gchunk = wid * KNCH + k

        @pl.when(gchunk < LASTC)
        def _():
            pltpu.async_copy(ei_ref.at[0, pl.ds(base, KCH)], rawr.at[d],
                             lsem)
            pltpu.async_copy(ei_ref.at[1, pl.ds(base, KCH)], rawc.at[d],
                             lsem)

        @pl.when(gchunk == LASTC)
        def _():
            pltpu.async_copy(ei_ref.at[0, pl.ds(base, TAIL)],
                             rawr.at[d, pl.ds(0, TAIL)], lsem)
            pltpu.async_copy(ei_ref.at[1, pl.ds(base, TAIL)],
                             rawc.at[d, pl.ds(0, TAIL)], lsem)

    def loads_wait(k, d):
        base = ebase + k * KCH
        gchunk = wid * KNCH + k

        @pl.when(gchunk < LASTC)
        def _():
            pltpu.make_async_copy(ei_ref.at[0, pl.ds(base, KCH)],
                                  rawr.at[d], lsem).wait()
            pltpu.make_async_copy(ei_ref.at[1, pl.ds(base, KCH)],
                                  rawc.at[d], lsem).wait()

        @pl.when(gchunk == LASTC)
        def _():
            pltpu.make_async_copy(ei_ref.at[0, pl.ds(base, TAIL)],
                                  rawr.at[d, pl.ds(0, TAIL)], lsem).wait()
            pltpu.make_async_copy(ei_ref.at[1, pl.ds(base, TAIL)],
                                  rawc.at[d, pl.ds(0, TAIL)], lsem).wait()

    def outs(k, d):
        rb = rbase + k * 8
        yield idxlo.at[d], row2_ref.at[pl.ds(rb, 8), :]
        yield idxc.at[d], col2_ref.at[pl.ds(rb, 8), :]

    loads_issue(0, 0)

    def chunk(k, carry):
        d = k & 1

        @pl.when(k + 1 < KNCH)
        def _():
            loads_issue(k + 1, 1 - d)
        loads_wait(k, d)

        @pl.when(k >= 2)
        def _():
            for src, dst in outs(k - 2, d):
                pltpu.make_async_copy(src, dst, osem).wait()
            for j in range(8):
                pltpu.make_async_copy(ones_v.at[j],
                                      deg_spmem.at[idxlo.at[d, j]],
                                      dsem).wait()
        base = ebase + k * KCH
        for j in range(8):
            for l in range(8):
                off = j * 128 + l * 16
                r = rawr[d, pl.ds(off, 16)]
                cc = rawc[d, pl.ds(off, 16)]
                ge = base + off + lax.iota(jnp.int32, 16)
                emask = ge < E
                trash = HIDX + (ge & 63)
                r2 = jnp.where(r >= HIDX, r + PSHIFT, r)
                c2 = jnp.where(cc >= HIDX, cc + PSHIFT, cc)
                r2 = jnp.where(emask, r2, trash)
                idxlo[d, j, pl.ds(l * 16, 16)] = r2
                idxc[d, j, pl.ds(l * 16, 16)] = jnp.where(emask, c2, trash)
        for src, dst in outs(k, d):
            pltpu.async_copy(src, dst, osem)
        for j in range(8):
            pltpu.async_copy(ones_v.at[j], deg_spmem.at[idxlo.at[d, j]],
                             dsem, add=True)
        return carry
    lax.fori_loop(0, KNCH, chunk, 0)
    for k in (KNCH - 2, KNCH - 1):
        d = k & 1
        for src, dst in outs(k, d):
            pltpu.make_async_copy(src, dst, osem).wait()
        for j in range(8):
            pltpu.make_async_copy(ones_v.at[j], deg_spmem.at[idxlo.at[d, j]],
                                  dsem).wait()
    plsc.subcore_barrier()
    pltpu.sync_copy(deg_spmem.at[pl.ds(s * RPS, RPS)],
                    deg_ref.at[c, pl.ds(s * RPS, RPS)])


def _k0(ei_pad):
    k = pl.kernel(
        _k0_body,
        out_type=(
            jax.ShapeDtypeStruct((EP128, 128), jnp.int32),
            jax.ShapeDtypeStruct((EP128, 128), jnp.int32),
            jax.ShapeDtypeStruct((2, N2), jnp.float32),
        ),
        mesh=_sc_mesh(),
        compiler_params=pltpu.CompilerParams(use_tc_tiling_on_sc=False),
        scratch_types=[
            pltpu.VMEM_SHARED((N2,), jnp.float32),
            pltpu.VMEM((2, KCH), jnp.int32),
            pltpu.VMEM((2, KCH), jnp.int32),
            pltpu.VMEM((2, 8, 128), jnp.int32),
            pltpu.VMEM((2, 8, 128), jnp.int32),
            pltpu.VMEM((8, 128), jnp.float32),
            pltpu.VMEM((RPS,), jnp.float32),
            pltpu.SemaphoreType.DMA,
            pltpu.SemaphoreType.DMA,
            pltpu.SemaphoreType.DMA,
        ],
    )
    return k(ei_pad)


DEPTH = 8
LAG = 4
IDEPTH = 4


def _edge_body(mode, nch, scaled_ref, row2_ref, col2_ref, out_ref,
               tab_spmem, tmp_spmem, idxr, idxc, msg, gsem, ssem, isem):
    c = lax.axis_index("c")
    s = lax.axis_index("s")
    if mode == "half":
        rrow = s * (nch * 8)
        rcol = rrow
        stage = c * N2 + s * RPS
    else:
        wid = c * 16 + s
        rrow = wid * (nch * 8)
        rcol = rrow
        stage = s * RPS
    pltpu.sync_copy(scaled_ref.at[pl.ds(stage, RPS)],
                    tmp_spmem.at[pl.ds(s * RPS, RPS)])
    pltpu.sync_copy(scaled_ref.at[pl.ds(stage, RPS)],
                    tab_spmem.at[pl.ds(s * RPS, RPS)])
    plsc.subcore_barrier()

    def idx_issue(k):
        di = k % IDEPTH
        pltpu.async_copy(row2_ref.at[pl.ds(rrow + k * 8, 8), :],
                         idxr.at[di], isem)
        pltpu.async_copy(col2_ref.at[pl.ds(rcol + k * 8, 8), :],
                         idxc.at[di], isem)

    def idx_wait(k):
        di = k % IDEPTH
        pltpu.make_async_copy(row2_ref.at[pl.ds(rrow + k * 8, 8), :],
                              idxr.at[di], isem).wait()
        pltpu.make_async_copy(col2_ref.at[pl.ds(rcol + k * 8, 8), :],
                              idxc.at[di], isem).wait()

    def swait():
        pltpu.make_async_copy(msg.at[0], tmp_spmem.at[idxc.at[0, 0]],
                              ssem).wait()

    def gwait():
        pltpu.make_async_copy(tab_spmem.at[idxr.at[0, 0]], msg.at[0],
                              gsem).wait()

    idx_issue(0)
    idx_issue(1)

    def chunk(k, carry):
        idx_wait(k)
        di = k % IDEPTH
        for j in range(8):
            d = j

            @pl.when(k >= 1)
            def _():
                swait()
            pltpu.async_copy(tab_spmem.at[idxr.at[di, j]], msg.at[d], gsem)

            def sct_issue():
                g2 = k * 8 + j - LAG
                d2 = g2 % DEPTH
                k2 = g2 // 8
                pltpu.make_async_copy(tab_spmem.at[idxr.at[0, 0]],
                                      msg.at[0], gsem).wait()
                pltpu.async_copy(msg.at[d2],
                                 tmp_spmem.at[idxc.at[k2 % IDEPTH, g2 % 8]],
                                 ssem, add=True)
            if j >= LAG:
                sct_issue()
            else:
                @pl.when(k >= 1)
                def _():
                    sct_issue()
        idx_issue(k + 2)
        return carry
    lax.fori_loop(0, nch - 2, chunk, 0)
    for k in (nch - 2, nch - 1):
        idx_wait(k)
        di = k % IDEPTH
        for j in range(8):
            d = j
            swait()
            pltpu.async_copy(tab_spmem.at[idxr.at[di, j]], msg.at[d], gsem)
            g2 = k * 8 + j - LAG
            gwait()
            pltpu.async_copy(msg.at[g2 % DEPTH],
                             tmp_spmem.at[idxc.at[(g2 // 8) % IDEPTH,
                                                  g2 % 8]],
                             ssem, add=True)
    for g in range(nch * 8 - LAG, nch * 8):
        gwait()
        pltpu.async_copy(msg.at[g % DEPTH],
                         tmp_spmem.at[idxc.at[(g // 8) % IDEPTH, g % 8]],
                         ssem, add=True)
    for _ in range(DEPTH):
        swait()
    plsc.subcore_barrier()
    pltpu.sync_copy(tmp_spmem.at[pl.ds(s * RPS, RPS)],
                    out_ref.at[c, pl.ds(s * RPS, RPS)])


def _edge_pass(scaled, row2, col2, mode):
    nch = (E_PAD // 16 // KCH) if mode == "half" else (E_PAD // 32 // KCH)
    k = pl.kernel(
        functools.partial(_edge_body, mode, nch),
        out_type=jax.ShapeDtypeStruct((2, N2, 16), jnp.float32),
        mesh=_sc_mesh(),
        compiler_params=pltpu.CompilerParams(use_tc_tiling_on_sc=False),
        scratch_types=[
            pltpu.VMEM_SHARED((N2, 16), jnp.float32),
            pltpu.VMEM_SHARED((N2, 16), jnp.float32),
            pltpu.VMEM((IDEPTH, 8, 128), jnp.int32),
            pltpu.VMEM((IDEPTH, 8, 128), jnp.int32),
            pltpu.VMEM((DEPTH, 128, 16), jnp.float32),
            pltpu.SemaphoreType.DMA,
            pltpu.SemaphoreType.DMA,
            pltpu.SemaphoreType.DMA,
        ],
    )
    return k(scaled, row2, col2)


def _dense1_body(x_ref, w_ref, b_ref, deg_ref, out0_ref, out1_ref, dinv_ref):
    i = pl.program_id(0)
    acc = jnp.dot(x_ref[...], w_ref[...],
                  preferred_element_type=jnp.float32) + b_ref[...][None, :]
    d = deg_ref[...]
    deg = 1.0 + d[:, 0:1] + d[:, 1:2]
    dinv = lax.rsqrt(deg)
    rows = i * 256 + lax.broadcasted_iota(jnp.int32, (256, 1), 0)
    valid = rows < HIDX
    sc = jnp.where(valid, dinv * acc, 0.0)
    out0_ref[...] = sc[:, :16]
    out1_ref[...] = sc[:, 16:]
    dinv_ref[...] = jnp.where(valid, dinv, 1.0)


def _dense1(x, w, b, degT_part):
    return pl.pallas_call(
        _dense1_body,
        grid=(HPAD // 256,),
        in_specs=[
            pl.BlockSpec((256, 200), lambda i: (i, 0)),
            pl.BlockSpec((200, 32), lambda i: (0, 0)),
            pl.BlockSpec((32,), lambda i: (0,)),
            pl.BlockSpec((256, 2), lambda i: (i, 0)),
        ],
        out_specs=[
            pl.BlockSpec((256, 16), lambda i: (i, 0)),
            pl.BlockSpec((256, 16), lambda i: (i, 0)),
            pl.BlockSpec((256, 1), lambda i: (i, 0)),
        ],
        out_shape=[
            jax.ShapeDtypeStruct((HPAD, 16), jnp.float32),
            jax.ShapeDtypeStruct((HPAD, 16), jnp.float32),
            jax.ShapeDtypeStruct((HPAD, 1), jnp.float32),
        ],
    )(x, w, b, degT_part)


def _dense2_body(t_ref, dinv_ref, w_ref, b_ref, out_ref):
    i = pl.program_id(0)
    t = jnp.concatenate([t_ref[0], t_ref[1]], axis=1)
    a = dinv_ref[...] * t
    a = jnp.where(a >= 0, a, 0.01 * a)
    b = jnp.where(i < 100, b_ref[0:1, :], b_ref[1:2, :])
    c2 = jnp.dot(a, w_ref[0],
                 preferred_element_type=jnp.float32) + b
    sc2 = dinv_ref[...] * c2
    rows = i * 256 + lax.broadcasted_iota(jnp.int32, (256, 1), 0)
    valid = (rows < HIDX) | ((rows >= HPAD) & (rows < HPAD + HIDX))
    out_ref[...] = jnp.where(valid, sc2, 0.0)


def _dense2(tmp1, dinv, w2s, b2s):
    return pl.pallas_call(
        _dense2_body,
        grid=(N2 // 256,),
        in_specs=[
            pl.BlockSpec((2, 256, 16), lambda i: (0, i, 0)),
            pl.BlockSpec((256, 1), lambda i: (i, 0)),
            pl.BlockSpec((1, 32, 16), lambda i: (i // 100, 0, 0)),
            pl.BlockSpec((2, 16), lambda i: (0, 0)),
        ],
        out_specs=pl.BlockSpec((256, 16), lambda i: (i, 0)),
        out_shape=jax.ShapeDtypeStruct((N2, 16), jnp.float32),
    )(tmp1, dinv, w2s, b2s)


def _softmax_body(t_ref, s_ref, dinv_ref, out_ref):
    t = t_ref[0] + t_ref[1] - s_ref[...]
    a = dinv_ref[...] * t
    a = jnp.where(a >= 0, a, 0.01 * a)
    lane = lax.broadcasted_iota(jnp.int32, (256, 16), 1)
    valid = lane < 7
    m = jnp.max(jnp.where(valid, a, -jnp.inf), axis=1, keepdims=True)
    e = jnp.where(valid, jnp.exp(a - m), 0.0)
    out_ref[...] = e / jnp.sum(e, axis=1, keepdims=True)


def _softmax(tmp2, scaled2, dinv):
    return pl.pallas_call(
        _softmax_body,
        grid=(HPAD // 256,),
        in_specs=[
            pl.BlockSpec((2, 256, 16), lambda i: (0, i, 0)),
            pl.BlockSpec((256, 16), lambda i: (i, 0)),
            pl.BlockSpec((256, 1), lambda i: (i, 0)),
        ],
        out_specs=pl.BlockSpec((256, 16), lambda i: (i, 0)),
        out_shape=jax.ShapeDtypeStruct((HPAD, 16), jnp.float32),
    )(tmp2, scaled2, dinv)


def kernel(x1, x2, edge_index, Wd1, bd1, Wg1, bg1, Wd2, bd2, Wg2, bg2):
    ei = edge_index.astype(jnp.int32)

    row2, col2, deg_parts = _k0(ei)
    degT = deg_parts.T

    wd1 = jnp.pad(Wd1, ((0, 0), (0, 2)))
    wg1 = jnp.pad(Wg1, ((0, 0), (0, 2)))
    b1d = jnp.pad(bd1, (0, 2))
    b1g = jnp.pad(bg1, (0, 2))
    h0a, h1a, dinva = _dense1(x1, wd1, b1d, degT[:HPAD])
    h0b, h1b, dinvb = _dense1(x2, wg1, b1g, degT[HPAD:])
    scaled1 = jnp.concatenate([h0a, h0b, h1a, h1b], axis=0)
    dinv = jnp.concatenate([dinva, dinvb], axis=0)

    tmp1 = _edge_pass(scaled1, row2, col2, "half")

    w2s = jnp.stack([jnp.pad(Wd2, ((0, 2), (0, 9))),
                     jnp.pad(Wg2, ((0, 2), (0, 9)))])
    b2s = jnp.stack([jnp.pad(bd2, (0, 9)), jnp.pad(bg2, (0, 9))])
    scaled2 = _dense2(tmp1, dinv, w2s, b2s)

    tmp2 = _edge_pass(scaled2, row2, col2, "split")

    out = _softmax(tmp2, scaled2, dinv)
    return out[:HIDX, :7]


# speedup vs baseline: 84.9701x; 1.2940x over previous
import functools

import jax
import jax.numpy as jnp
from jax import lax
from jax.experimental import pallas as pl
from jax.experimental.pallas import tpu as pltpu
from jax.experimental.pallas import tpu_sc as plsc

HIDX = 25453
HPAD = 25600
N2 = 2 * HPAD
E = 1628992
KCH = 1024
NTILES = 32
KNCH = 50
E_PAD = NTILES * KNCH * KCH
EP128 = E_PAD // 128
RPS = N2 // 16
PSHIFT = HPAD - HIDX

ECH = 512
DEPTH = 6
IDEPTH = 2 * DEPTH
LAG = 2


def _sc_mesh():
    return plsc.VectorSubcoreMesh(core_axis_name="c", subcore_axis_name="s")


def _k0_body(ei_ref, row2_ref, col2_ref, deg_ref,
             deg_spmem, rawr, rawc, idxlo, idxc, ones_v, zb,
             lsem, osem, dsem):
    c = lax.axis_index("c")
    s = lax.axis_index("s")
    wid = c * 16 + s

    def zf(i, carry):
        zb[pl.ds(i * 16, 16)] = jnp.zeros((16,), jnp.float32)
        return carry
    lax.fori_loop(0, RPS // 16, zf, 0)
    for l in range(8):
        for q in range(8):
            ones_v[l, pl.ds(q * 16, 16)] = jnp.ones((16,), jnp.float32)
    pltpu.sync_copy(zb, deg_spmem.at[pl.ds(s * RPS, RPS)])
    plsc.subcore_barrier()

    ebase = wid * (KNCH * KCH)
    rbase = wid * (KNCH * 8)

    LASTC = E // KCH
    TAIL = E - LASTC * KCH

    def loads_issue(k, d):
        base = ebase + k * KCH
        gchunk = wid * KNCH + k

        @pl.when(gchunk < LASTC)
        def _():
            pltpu.async_copy(ei_ref.at[0, pl.ds(base, KCH)], rawr.at[d],
                             lsem)
            pltpu.async_copy(ei_ref.at[1, pl.ds(base, KCH)], rawc.at[d],
                             lsem)

        @pl.when(gchunk == LASTC)
        def _():
            pltpu.async_copy(ei_ref.at[0, pl.ds(base, TAIL)],
                             rawr.at[d, pl.ds(0, TAIL)], lsem)
            pltpu.async_copy(ei_ref.at[1, pl.ds(base, TAIL)],
                             rawc.at[d, pl.ds(0, TAIL)], lsem)

    def loads_wait(k, d):
        base = ebase + k * KCH
        gchunk = wid * KNCH + k

        @pl.when(gchunk < LASTC)
        def _():
            pltpu.make_async_copy(ei_ref.at[0, pl.ds(base, KCH)],
                                  rawr.at[d], lsem).wait()
            pltpu.make_async_copy(ei_ref.at[1, pl.ds(base, KCH)],
                                  rawc.at[d], lsem).wait()

        @pl.when(gchunk == LASTC)
        def _():
            pltpu.make_async_copy(ei_ref.at[0, pl.ds(base, TAIL)],
                                  rawr.at[d, pl.ds(0, TAIL)], lsem).wait()
            pltpu.make_async_copy(ei_ref.at[1, pl.ds(base, TAIL)],
                                  rawc.at[d, pl.ds(0, TAIL)], lsem).wait()

    def outs(k, d):
        rb = rbase + k * 8
        yield idxlo.at[d], row2_ref.at[pl.ds(rb, 8), :]
        yield idxc.at[d], col2_ref.at[pl.ds(rb, 8), :]

    loads_issue(0, 0)

    def chunk(k, carry):
        d = k & 1

        @pl.when(k + 1 < KNCH)
        def _():
            loads_issue(k + 1, 1 - d)
        loads_wait(k, d)

        @pl.when(k >= 2)
        def _():
            for src, dst in outs(k - 2, d):
                pltpu.make_async_copy(src, dst, osem).wait()
            for j in range(8):
                pltpu.make_async_copy(ones_v.at[j],
                                      deg_spmem.at[idxlo.at[d, j]],
                                      dsem).wait()
        base = ebase + k * KCH
        for j in range(8):
            for l in range(8):
                off = j * 128 + l * 16
                r = rawr[d, pl.ds(off, 16)]
                cc = rawc[d, pl.ds(off, 16)]
                ge = base + off + lax.iota(jnp.int32, 16)
                emask = ge < E
                trash = HIDX + (ge & 63)
                r2 = jnp.where(r >= HIDX, r + PSHIFT, r)
                c2 = jnp.where(cc >= HIDX, cc + PSHIFT, cc)
                r2 = jnp.where(emask, r2, trash)
                idxlo[d, j, pl.ds(l * 16, 16)] = r2
                idxc[d, j, pl.ds(l * 16, 16)] = jnp.where(emask, c2, trash)
        for src, dst in outs(k, d):
            pltpu.async_copy(src, dst, osem)
        for j in range(8):
            pltpu.async_copy(ones_v.at[j], deg_spmem.at[idxlo.at[d, j]],
                             dsem, add=True)
        return carry
    lax.fori_loop(0, KNCH, chunk, 0)
    for k in (KNCH - 2, KNCH - 1):
        d = k & 1
        for src, dst in outs(k, d):
            pltpu.make_async_copy(src, dst, osem).wait()
        for j in range(8):
            pltpu.make_async_copy(ones_v.at[j], deg_spmem.at[idxlo.at[d, j]],
                                  dsem).wait()
    plsc.subcore_barrier()
    pltpu.sync_copy(deg_spmem.at[pl.ds(s * RPS, RPS)],
                    deg_ref.at[c, pl.ds(s * RPS, RPS)])


def _k0(ei_pad):
    k = pl.kernel(
        _k0_body,
        out_type=(
            jax.ShapeDtypeStruct((EP128, 128), jnp.int32),
            jax.ShapeDtypeStruct((EP128, 128), jnp.int32),
            jax.ShapeDtypeStruct((2, N2), jnp.float32),
        ),
        mesh=_sc_mesh(),
        compiler_params=pltpu.CompilerParams(use_tc_tiling_on_sc=False),
        scratch_types=[
            pltpu.VMEM_SHARED((N2,), jnp.float32),
            pltpu.VMEM((2, KCH), jnp.int32),
            pltpu.VMEM((2, KCH), jnp.int32),
            pltpu.VMEM((2, 8, 128), jnp.int32),
            pltpu.VMEM((2, 8, 128), jnp.int32),
            pltpu.VMEM((8, 128), jnp.float32),
            pltpu.VMEM((RPS,), jnp.float32),
            pltpu.SemaphoreType.DMA,
            pltpu.SemaphoreType.DMA,
            pltpu.SemaphoreType.DMA,
        ],
    )
    return k(ei_pad)


DEPTH = 8
LAG = 4
IDEPTH = 4


def _edge_body(mode, nch, scaled_ref, row2_ref, col2_ref, out_ref,
               tab_spmem, tmp_spmem, idxr, idxc, msg, gsem, ssem, isem):
    c = lax.axis_index("c")
    s = lax.axis_index("s")
    if mode == "half":
        rrow = s * (nch * 8)
        rcol = rrow
        stage = c * N2 + s * RPS
    else:
        wid = c * 16 + s
        rrow = wid * (nch * 8)
        rcol = rrow
        stage = s * RPS
    pltpu.sync_copy(scaled_ref.at[pl.ds(stage, RPS)],
                    tmp_spmem.at[pl.ds(s * RPS, RPS)])
    pltpu.sync_copy(scaled_ref.at[pl.ds(stage, RPS)],
                    tab_spmem.at[pl.ds(s * RPS, RPS)])
    plsc.subcore_barrier()

    def idx_issue(k):
        di = k % IDEPTH
        pltpu.async_copy(row2_ref.at[pl.ds(rrow + k * 8, 8), :],
                         idxr.at[di], isem)
        pltpu.async_copy(col2_ref.at[pl.ds(rcol + k * 8, 8), :],
                         idxc.at[di], isem)

    def idx_wait(k):
        di = k % IDEPTH
        pltpu.make_async_copy(row2_ref.at[pl.ds(rrow + k * 8, 8), :],
                              idxr.at[di], isem).wait()
        pltpu.make_async_copy(col2_ref.at[pl.ds(rcol + k * 8, 8), :],
                              idxc.at[di], isem).wait()

    def swait():
        pltpu.make_async_copy(msg.at[0], tmp_spmem.at[idxc.at[0, 0]],
                              ssem).wait()

    def gwait():
        pltpu.make_async_copy(tab_spmem.at[idxr.at[0, 0]], msg.at[0],
                              gsem).wait()

    idx_issue(0)
    idx_issue(1)

    def chunk(k, carry):
        idx_wait(k)
        di = k % IDEPTH
        for j in range(8):
            d = j

            @pl.when(k >= 1)
            def _():
                swait()
            pltpu.async_copy(tab_spmem.at[idxr.at[di, j]], msg.at[d], gsem)

            def sct_issue():
                g2 = k * 8 + j - LAG
                d2 = g2 % DEPTH
                k2 = g2 // 8
                pltpu.make_async_copy(tab_spmem.at[idxr.at[0, 0]],
                                      msg.at[0], gsem).wait()
                pltpu.async_copy(msg.at[d2],
                                 tmp_spmem.at[idxc.at[k2 % IDEPTH, g2 % 8]],
                                 ssem, add=True)
            if j >= LAG:
                sct_issue()
            else:
                @pl.when(k >= 1)
                def _():
                    sct_issue()
        idx_issue(k + 2)
        return carry
    lax.fori_loop(0, nch - 2, chunk, 0)
    for k in (nch - 2, nch - 1):
        idx_wait(k)
        di = k % IDEPTH
        for j in range(8):
            d = j
            swait()
            pltpu.async_copy(tab_spmem.at[idxr.at[di, j]], msg.at[d], gsem)
            g2 = k * 8 + j - LAG
            gwait()
            pltpu.async_copy(msg.at[g2 % DEPTH],
                             tmp_spmem.at[idxc.at[(g2 // 8) % IDEPTH,
                                                  g2 % 8]],
                             ssem, add=True)
    for g in range(nch * 8 - LAG, nch * 8):
        gwait()
        pltpu.async_copy(msg.at[g % DEPTH],
                         tmp_spmem.at[idxc.at[(g // 8) % IDEPTH, g % 8]],
                         ssem, add=True)
    for _ in range(DEPTH):
        swait()
    plsc.subcore_barrier()
    pltpu.sync_copy(tmp_spmem.at[pl.ds(s * RPS, RPS)],
                    out_ref.at[c, pl.ds(s * RPS, RPS)])


def _edge_pass(scaled, row2, col2, mode):
    nch = (E_PAD // 16 // KCH) if mode == "half" else (E_PAD // 32 // KCH)
    k = pl.kernel(
        functools.partial(_edge_body, mode, nch),
        out_type=jax.ShapeDtypeStruct((2, N2, 16), jnp.float32),
        mesh=_sc_mesh(),
        compiler_params=pltpu.CompilerParams(use_tc_tiling_on_sc=False),
        scratch_types=[
            pltpu.VMEM_SHARED((N2, 16), jnp.float32),
            pltpu.VMEM_SHARED((N2, 16), jnp.float32),
            pltpu.VMEM((IDEPTH, 8, 128), jnp.int32),
            pltpu.VMEM((IDEPTH, 8, 128), jnp.int32),
            pltpu.VMEM((DEPTH, 128, 16), jnp.float32),
            pltpu.SemaphoreType.DMA,
            pltpu.SemaphoreType.DMA,
            pltpu.SemaphoreType.DMA,
        ],
    )
    return k(scaled, row2, col2)


def _dense1_body(x_ref, w_ref, b_ref, deg_ref, out0_ref, out1_ref, dinv_ref):
    i = pl.program_id(0)
    acc = jnp.dot(x_ref[...], w_ref[...],
                  preferred_element_type=jnp.float32) + b_ref[...][None, :]
    d = deg_ref[...]
    deg = 1.0 + d[:, 0:1] + d[:, 1:2]
    dinv = lax.rsqrt(deg)
    rows = i * 1024 + lax.broadcasted_iota(jnp.int32, (1024, 1), 0)
    valid = rows < HIDX
    sc = jnp.where(valid, dinv * acc, 0.0)
    out0_ref[...] = sc[:, :16]
    out1_ref[...] = sc[:, 16:]
    dinv_ref[...] = jnp.where(valid, dinv, 1.0)


def _dense1(x, w, b, degT_part):
    return pl.pallas_call(
        _dense1_body,
        grid=(HPAD // 1024,),
        in_specs=[
            pl.BlockSpec((1024, 200), lambda i: (i, 0)),
            pl.BlockSpec((200, 32), lambda i: (0, 0)),
            pl.BlockSpec((32,), lambda i: (0,)),
            pl.BlockSpec((1024, 2), lambda i: (i, 0)),
        ],
        out_specs=[
            pl.BlockSpec((1024, 16), lambda i: (i, 0)),
            pl.BlockSpec((1024, 16), lambda i: (i, 0)),
            pl.BlockSpec((1024, 1), lambda i: (i, 0)),
        ],
        out_shape=[
            jax.ShapeDtypeStruct((HPAD, 16), jnp.float32),
            jax.ShapeDtypeStruct((HPAD, 16), jnp.float32),
            jax.ShapeDtypeStruct((HPAD, 1), jnp.float32),
        ],
    )(x, w, b, degT_part)


def _dense2_body(t_ref, dinv_ref, w_ref, b_ref, out_ref):
    i = pl.program_id(0)
    t = jnp.concatenate([t_ref[0], t_ref[1]], axis=1)
    a = dinv_ref[...] * t
    a = jnp.where(a >= 0, a, 0.01 * a)
    b = jnp.where(i < 25, b_ref[0:1, :], b_ref[1:2, :])
    c2 = jnp.dot(a, w_ref[0],
                 preferred_element_type=jnp.float32) + b
    sc2 = dinv_ref[...] * c2
    rows = i * 1024 + lax.broadcasted_iota(jnp.int32, (1024, 1), 0)
    valid = (rows < HIDX) | ((rows >= HPAD) & (rows < HPAD + HIDX))
    out_ref[...] = jnp.where(valid, sc2, 0.0)


def _dense2(tmp1, dinv, w2s, b2s):
    return pl.pallas_call(
        _dense2_body,
        grid=(N2 // 1024,),
        in_specs=[
            pl.BlockSpec((2, 1024, 16), lambda i: (0, i, 0)),
            pl.BlockSpec((1024, 1), lambda i: (i, 0)),
            pl.BlockSpec((1, 32, 16), lambda i: (i // 25, 0, 0)),
            pl.BlockSpec((2, 16), lambda i: (0, 0)),
        ],
        out_specs=pl.BlockSpec((1024, 16), lambda i: (i, 0)),
        out_shape=jax.ShapeDtypeStruct((N2, 16), jnp.float32),
    )(tmp1, dinv, w2s, b2s)


def _softmax_body(t_ref, s_ref, dinv_ref, out_ref):
    t = t_ref[0] + t_ref[1] - s_ref[...]
    a = dinv_ref[...] * t
    a = jnp.where(a >= 0, a, 0.01 * a)
    lane = lax.broadcasted_iota(jnp.int32, (1024, 16), 1)
    valid = lane < 7
    m = jnp.max(jnp.where(valid, a, -jnp.inf), axis=1, keepdims=True)
    e = jnp.where(valid, jnp.exp(a - m), 0.0)
    out_ref[...] = e / jnp.sum(e, axis=1, keepdims=True)


def _softmax(tmp2, scaled2, dinv):
    return pl.pallas_call(
        _softmax_body,
        grid=(HPAD // 1024,),
        in_specs=[
            pl.BlockSpec((2, 1024, 16), lambda i: (0, i, 0)),
            pl.BlockSpec((1024, 16), lambda i: (i, 0)),
            pl.BlockSpec((1024, 1), lambda i: (i, 0)),
        ],
        out_specs=pl.BlockSpec((1024, 16), lambda i: (i, 0)),
        out_shape=jax.ShapeDtypeStruct((HPAD, 16), jnp.float32),
    )(tmp2, scaled2, dinv)


def kernel(x1, x2, edge_index, Wd1, bd1, Wg1, bg1, Wd2, bd2, Wg2, bg2):
    ei = edge_index.astype(jnp.int32)

    row2, col2, deg_parts = _k0(ei)
    degT = deg_parts.T

    wd1 = jnp.pad(Wd1, ((0, 0), (0, 2)))
    wg1 = jnp.pad(Wg1, ((0, 0), (0, 2)))
    b1d = jnp.pad(bd1, (0, 2))
    b1g = jnp.pad(bg1, (0, 2))
    h0a, h1a, dinva = _dense1(x1, wd1, b1d, degT[:HPAD])
    h0b, h1b, dinvb = _dense1(x2, wg1, b1g, degT[HPAD:])
    scaled1 = jnp.concatenate([h0a, h0b, h1a, h1b], axis=0)
    dinv = jnp.concatenate([dinva, dinvb], axis=0)

    tmp1 = _edge_pass(scaled1, row2, col2, "half")

    w2s = jnp.stack([jnp.pad(Wd2, ((0, 2), (0, 9))),
                     jnp.pad(Wg2, ((0, 2), (0, 9)))])
    b2s = jnp.stack([jnp.pad(bd2, (0, 9)), jnp.pad(bg2, (0, 9))])
    scaled2 = _dense2(tmp1, dinv, w2s, b2s)

    tmp2 = _edge_pass(scaled2, row2, col2, "split")

    out = _softmax(tmp2, scaled2, dinv)
    return out[:HIDX, :7]
